# baseline (speedup 1.0000x reference)
import functools

import jax
import jax.numpy as jnp
from jax import lax
from jax.experimental import pallas as pl
from jax.experimental.pallas import tpu as pltpu
from jax.experimental.pallas import tpu_sc as plsc

N = 10000
E = 320000
ATOM_F = 128
EDGE_F = 16
OUT_F = 128

BN = 1000
EB = 2000
KW = OUT_F // 2

_NC = 2
_NS = 16
_NW = _NC * _NS
_PER_W = E // _NW
_K = 80
_CHUNKS = _PER_W // _K


def _node_body(z_ref, emb_ref, w_ref, h_ref, gi_ref, gj_ref):
    zm1 = z_ref[...] - 1
    col = lax.broadcasted_iota(jnp.int32, (BN, ATOM_F), 1)
    onehot = (zm1 == col).astype(jnp.float32)
    h = jnp.dot(onehot, emb_ref[...], preferred_element_type=jnp.float32)
    h_ref[...] = h
    wi = w_ref[:, 0:ATOM_F]
    wj = w_ref[:, ATOM_F:2 * ATOM_F]
    dn = (((1,), (1,)), ((), ()))
    gi = lax.dot_general(h, wi, dn, preferred_element_type=jnp.float32)
    gj = lax.dot_general(h, wj, dn, preferred_element_type=jnp.float32)
    gi_ref[...] = gi.astype(jnp.bfloat16)
    gj_ref[...] = gj.astype(jnp.bfloat16)


def _node_call(z2d, emb_pad, w):
    return pl.pallas_call(
        _node_body,
        grid=(N // BN,),
        in_specs=[
            pl.BlockSpec((BN, 1), lambda i: (i, 0)),
            pl.BlockSpec((ATOM_F, ATOM_F), lambda i: (0, 0)),
            pl.BlockSpec((OUT_F, 2 * ATOM_F + EDGE_F), lambda i: (0, 0)),
        ],
        out_specs=[pl.BlockSpec((BN, ATOM_F), lambda i: (i, 0))] * 3,
        out_shape=[jax.ShapeDtypeStruct((N, ATOM_F), jnp.float32),
                   jax.ShapeDtypeStruct((N, ATOM_F), jnp.bfloat16),
                   jax.ShapeDtypeStruct((N, ATOM_F), jnp.bfloat16)],
    )(z2d, emb_pad, w)


@functools.lru_cache(maxsize=None)
def _make_edge_gather():
    mesh = plsc.VectorSubcoreMesh(core_axis_name="c", subcore_axis_name="s")

    @functools.partial(
        pl.kernel,
        mesh=mesh,
        out_type=[jax.ShapeDtypeStruct((E, KW), jnp.int32)] * 2,
        compiler_params=pltpu.CompilerParams(use_tc_tiling_on_sc=False),
        scratch_types=[
            pltpu.VMEM((_PER_W,), jnp.int32),
            pltpu.VMEM((_PER_W,), jnp.int32),
            pltpu.VMEM((_K, KW), jnp.int32),
            pltpu.VMEM((_K, KW), jnp.int32),
            pltpu.VMEM((_K, KW), jnp.int32),
            pltpu.VMEM((_K, KW), jnp.int32),
            pltpu.SemaphoreType.DMA,
            pltpu.SemaphoreType.DMA,
            pltpu.SemaphoreType.DMA,
            pltpu.SemaphoreType.DMA,
        ],
    )
    def _edge_gather(gi_hbm, gj_hbm, ii_hbm, jj_hbm, si_hbm, sj_hbm,
                     ii_v, jj_v, ri0, ri1, rj0, rj1,
                     smi0, smi1, smj0, smj1):
        wid = lax.axis_index("s") * _NC + lax.axis_index("c")
        wbase = wid * _PER_W
        pltpu.sync_copy(ii_hbm.at[pl.ds(wbase, _PER_W)], ii_v)
        pltpu.sync_copy(jj_hbm.at[pl.ds(wbase, _PER_W)], jj_v)

        def issue(c, ri, rj, smi, smj):
            sl = pl.ds(c * _K, _K)
            pltpu.async_copy(gi_hbm.at[ii_v.at[sl]], ri, smi)
            pltpu.async_copy(gj_hbm.at[jj_v.at[sl]], rj, smj)

        def wait(ri, rj, smi, smj):
            pltpu.make_async_copy(gi_hbm.at[ii_v.at[pl.ds(0, _K)]], ri, smi).wait()
            pltpu.make_async_copy(gj_hbm.at[jj_v.at[pl.ds(0, _K)]], rj, smj).wait()

        def store(c, ri, rj):
            sl = pl.ds(wbase + c * _K, _K)
            pltpu.sync_copy(ri, si_hbm.at[sl])
            pltpu.sync_copy(rj, sj_hbm.at[sl])

        issue(0, ri0, rj0, smi0, smj0)

        def pair_body(t, carry):
            c0 = 2 * t
            issue(c0 + 1, ri1, rj1, smi1, smj1)
            wait(ri0, rj0, smi0, smj0)
            store(c0, ri0, rj0)
            issue(c0 + 2, ri0, rj0, smi0, smj0)
            wait(ri1, rj1, smi1, smj1)
            store(c0 + 1, ri1, rj1)
            return carry

        lax.fori_loop(0, (_CHUNKS - 1) // 2, pair_body, 0)
        wait(ri0, rj0, smi0, smj0)
        store(_CHUNKS - 1, ri0, rj0)

    return _edge_gather


def _edge_out_body(si_ref, sj_ref, rbf_ref, wr_ref, b_ref, o_ref):
    dn = (((1,), (1,)), ((), ()))
    t = lax.dot_general(rbf_ref[...], wr_ref[...], dn,
                        preferred_element_type=jnp.float32)
    x = (si_ref[...].astype(jnp.float32) + sj_ref[...].astype(jnp.float32)
         + t + b_ref[...])
    o_ref[...] = x / (1.0 + jnp.exp(-x))


def _edge_out_call(si, sj, rbf, wr, b2d):
    return pl.pallas_call(
        _edge_out_body,
        grid=(E // EB,),
        in_specs=[
            pl.BlockSpec((EB, OUT_F), lambda i: (i, 0)),
            pl.BlockSpec((EB, OUT_F), lambda i: (i, 0)),
            pl.BlockSpec((EB, EDGE_F), lambda i: (i, 0)),
            pl.BlockSpec((OUT_F, EDGE_F), lambda i: (0, 0)),
            pl.BlockSpec((1, OUT_F), lambda i: (0, 0)),
        ],
        out_specs=pl.BlockSpec((EB, OUT_F), lambda i: (i, 0)),
        out_shape=jax.ShapeDtypeStruct((E, OUT_F), jnp.float32),
    )(si, sj, rbf, wr, b2d)


def _pack_i32(x_bf16):
    n = x_bf16.shape[0]
    return lax.bitcast_convert_type(x_bf16.reshape(n, KW, 2), jnp.int32)


def _unpack_bf16(x_i32):
    n = x_i32.shape[0]
    return lax.bitcast_convert_type(x_i32, jnp.bfloat16).reshape(n, OUT_F)


def kernel(z, rbf, idx_i, idx_j, emb, W, b):
    z2d = z.astype(jnp.int32).reshape(N, 1)
    emb_pad = jnp.pad(emb, ((0, ATOM_F - emb.shape[0]), (0, 0)))
    h, gi_bf, gj_bf = _node_call(z2d, emb_pad, W)
    si32, sj32 = _make_edge_gather()(
        _pack_i32(gi_bf), _pack_i32(gj_bf),
        idx_i.astype(jnp.int32), idx_j.astype(jnp.int32))
    wr = lax.slice(W, (0, 2 * ATOM_F), (OUT_F, 2 * ATOM_F + EDGE_F))
    m_ij = _edge_out_call(_unpack_bf16(si32), _unpack_bf16(sj32),
                          rbf, wr, b.reshape(1, OUT_F))
    return (h, m_ij)


# speedup vs baseline: 4.3471x; 4.3471x over previous
import functools

import jax
import jax.numpy as jnp
from jax import lax
from jax.experimental import pallas as pl
from jax.experimental.pallas import tpu as pltpu
from jax.experimental.pallas import tpu_sc as plsc

N = 10000
E = 320000
ATOM_F = 128
EDGE_F = 16
OUT_F = 128

BN = 1000
EB = 2000

_NC = 2
_NS = 16
_NW = _NC * _NS
_PER_W = E // _NW
_K = 80
_CHUNKS = _PER_W // _K


def _node_body(z_ref, emb_ref, w_ref, h_ref, gi_ref, gj_ref):
    zm1 = z_ref[...] - 1
    col = lax.broadcasted_iota(jnp.int32, (BN, ATOM_F), 1)
    onehot = (zm1 == col).astype(jnp.float32)
    h = jnp.dot(onehot, emb_ref[...], preferred_element_type=jnp.float32)
    h_ref[...] = h
    wi = w_ref[:, 0:ATOM_F]
    wj = w_ref[:, ATOM_F:2 * ATOM_F]
    dn = (((1,), (1,)), ((), ()))
    gi_ref[...] = lax.dot_general(h, wi, dn, preferred_element_type=jnp.float32)
    gj_ref[...] = lax.dot_general(h, wj, dn, preferred_element_type=jnp.float32)


def _node_call(z2d, emb_pad, w):
    return pl.pallas_call(
        _node_body,
        grid=(N // BN,),
        in_specs=[
            pl.BlockSpec((BN, 1), lambda i: (i, 0)),
            pl.BlockSpec((ATOM_F, ATOM_F), lambda i: (0, 0)),
            pl.BlockSpec((OUT_F, 2 * ATOM_F + EDGE_F), lambda i: (0, 0)),
        ],
        out_specs=[pl.BlockSpec((BN, ATOM_F), lambda i: (i, 0))] * 3,
        out_shape=[jax.ShapeDtypeStruct((N, ATOM_F), jnp.float32)] * 3,
    )(z2d, emb_pad, w)


@functools.lru_cache(maxsize=None)
def _make_edge_gather():
    mesh = plsc.VectorSubcoreMesh(core_axis_name="c", subcore_axis_name="s")

    @functools.partial(
        pl.kernel,
        mesh=mesh,
        out_type=jax.ShapeDtypeStruct((E, OUT_F), jnp.float32),
        scratch_types=[
            pltpu.VMEM((_PER_W,), jnp.int32),
            pltpu.VMEM((_PER_W,), jnp.int32),
            pltpu.VMEM((_K, OUT_F), jnp.float32),
            pltpu.VMEM((_K, OUT_F), jnp.float32),
            pltpu.VMEM((_K, OUT_F), jnp.float32),
            pltpu.VMEM((_K, OUT_F), jnp.float32),
            pltpu.SemaphoreType.DMA,
            pltpu.SemaphoreType.DMA,
            pltpu.SemaphoreType.DMA,
            pltpu.SemaphoreType.DMA,
        ],
    )
    def _edge_gather(gi_hbm, gj_hbm, ii_hbm, jj_hbm, out_hbm,
                     ii_v, jj_v, ri0, ri1, rj0, rj1,
                     smi0, smi1, smj0, smj1):
        wid = lax.axis_index("s") * _NC + lax.axis_index("c")
        wbase = wid * _PER_W
        pltpu.sync_copy(ii_hbm.at[pl.ds(wbase, _PER_W)], ii_v)
        pltpu.sync_copy(jj_hbm.at[pl.ds(wbase, _PER_W)], jj_v)

        def issue(c, ri, rj, smi, smj):
            sl = pl.ds(c * _K, _K)
            pltpu.async_copy(gi_hbm.at[ii_v.at[sl]], ri, smi)
            pltpu.async_copy(gj_hbm.at[jj_v.at[sl]], rj, smj)

        def wait(ri, rj, smi, smj):
            pltpu.make_async_copy(gi_hbm.at[ii_v.at[pl.ds(0, _K)]], ri, smi).wait()
            pltpu.make_async_copy(gj_hbm.at[jj_v.at[pl.ds(0, _K)]], rj, smj).wait()

        def add_store(c, ri, rj):
            def row_add(r, rcarry):
                for cb in range(OUT_F // 16):
                    sl = pl.ds(cb * 16, 16)
                    ri[r, sl] = ri[r, sl] + rj[r, sl]
                return rcarry

            lax.fori_loop(0, _K, row_add, 0)
            pltpu.sync_copy(ri, out_hbm.at[pl.ds(wbase + c * _K, _K)])

        issue(0, ri0, rj0, smi0, smj0)

        def pair_body(t, carry):
            c0 = 2 * t
            issue(c0 + 1, ri1, rj1, smi1, smj1)
            wait(ri0, rj0, smi0, smj0)
            add_store(c0, ri0, rj0)
            issue(c0 + 2, ri0, rj0, smi0, smj0)
            wait(ri1, rj1, smi1, smj1)
            add_store(c0 + 1, ri1, rj1)
            return carry

        lax.fori_loop(0, (_CHUNKS - 1) // 2, pair_body, 0)
        wait(ri0, rj0, smi0, smj0)
        add_store(_CHUNKS - 1, ri0, rj0)

    return _edge_gather


def _edge_out_body(s_ref, rbf_ref, wr_ref, b_ref, o_ref):
    dn = (((1,), (1,)), ((), ()))
    t = lax.dot_general(rbf_ref[...], wr_ref[...], dn,
                        preferred_element_type=jnp.float32)
    x = s_ref[...] + t + b_ref[...]
    o_ref[...] = x / (1.0 + jnp.exp(-x))


def _edge_out_call(s, rbf, wr, b2d):
    return pl.pallas_call(
        _edge_out_body,
        grid=(E // EB,),
        in_specs=[
            pl.BlockSpec((EB, OUT_F), lambda i: (i, 0)),
            pl.BlockSpec((EB, EDGE_F), lambda i: (i, 0)),
            pl.BlockSpec((OUT_F, EDGE_F), lambda i: (0, 0)),
            pl.BlockSpec((1, OUT_F), lambda i: (0, 0)),
        ],
        out_specs=pl.BlockSpec((EB, OUT_F), lambda i: (i, 0)),
        out_shape=jax.ShapeDtypeStruct((E, OUT_F), jnp.float32),
    )(s, rbf, wr, b2d)


def kernel(z, rbf, idx_i, idx_j, emb, W, b):
    z2d = z.astype(jnp.int32).reshape(N, 1)
    emb_pad = jnp.pad(emb, ((0, ATOM_F - emb.shape[0]), (0, 0)))
    h, gi, gj = _node_call(z2d, emb_pad, W)
    s = _make_edge_gather()(gi, gj, idx_i.astype(jnp.int32),
                            idx_j.astype(jnp.int32))
    wr = lax.slice(W, (0, 2 * ATOM_F), (OUT_F, 2 * ATOM_F + EDGE_F))
    m_ij = _edge_out_call(s, rbf, wr, b.reshape(1, OUT_F))
    return (h, m_ij)


# speedup vs baseline: 4.5003x; 1.0352x over previous
import functools

import jax
import jax.numpy as jnp
from jax import lax
from jax.experimental import pallas as pl
from jax.experimental.pallas import tpu as pltpu
from jax.experimental.pallas import tpu_sc as plsc

N = 10000
E = 320000
ATOM_F = 128
EDGE_F = 16
OUT_F = 128

BN = 1000
EB = 2000

_NC = 2
_NS = 16
_NW = _NC * _NS
_NSPLIT = 2
_ESPLIT = E // _NSPLIT
_PER_W = _ESPLIT // _NW
_K = 40
_CHUNKS = _PER_W // _K
_NB = _ESPLIT // EB


def _node_body(z_ref, emb_ref, w_ref, h_ref, gi_ref, gj_ref):
    zm1 = z_ref[...] - 1
    col = lax.broadcasted_iota(jnp.int32, (BN, ATOM_F), 1)
    onehot = (zm1 == col).astype(jnp.float32)
    h = jnp.dot(onehot, emb_ref[...], preferred_element_type=jnp.float32)
    h_ref[...] = h
    wi = w_ref[:, 0:ATOM_F]
    wj = w_ref[:, ATOM_F:2 * ATOM_F]
    dn = (((1,), (1,)), ((), ()))
    gi_ref[...] = lax.dot_general(h, wi, dn, preferred_element_type=jnp.float32)
    gj_ref[...] = lax.dot_general(h, wj, dn, preferred_element_type=jnp.float32)


def _node_call(z2d, emb_pad, w):
    return pl.pallas_call(
        _node_body,
        grid=(N // BN,),
        in_specs=[
            pl.BlockSpec((BN, 1), lambda i: (i, 0)),
            pl.BlockSpec((ATOM_F, ATOM_F), lambda i: (0, 0)),
            pl.BlockSpec((OUT_F, 2 * ATOM_F + EDGE_F), lambda i: (0, 0)),
        ],
        out_specs=[pl.BlockSpec((BN, ATOM_F), lambda i: (i, 0))] * 3,
        out_shape=[jax.ShapeDtypeStruct((N, ATOM_F), jnp.float32)] * 3,
    )(z2d, emb_pad, w)


@functools.lru_cache(maxsize=None)
def _make_edge_gather(part):
    mesh = plsc.VectorSubcoreMesh(core_axis_name="c", subcore_axis_name="s")
    ebase = part * _ESPLIT

    @functools.partial(
        pl.kernel,
        mesh=mesh,
        out_type=jax.ShapeDtypeStruct((_ESPLIT, OUT_F), jnp.float32),
        scratch_types=[
            pltpu.VMEM((_PER_W,), jnp.int32),
            pltpu.VMEM((_PER_W,), jnp.int32),
            pltpu.VMEM((_K, OUT_F), jnp.float32),
            pltpu.VMEM((_K, OUT_F), jnp.float32),
            pltpu.VMEM((_K, OUT_F), jnp.float32),
            pltpu.VMEM((_K, OUT_F), jnp.float32),
            pltpu.SemaphoreType.DMA,
            pltpu.SemaphoreType.DMA,
            pltpu.SemaphoreType.DMA,
            pltpu.SemaphoreType.DMA,
        ],
    )
    def _edge_gather(gi_hbm, gj_hbm, ii_hbm, jj_hbm, out_hbm,
                     ii_v, jj_v, ri0, ri1, rj0, rj1,
                     smi0, smi1, smj0, smj1):
        wid = lax.axis_index("s") * _NC + lax.axis_index("c")
        wbase = wid * _PER_W
        pltpu.sync_copy(ii_hbm.at[pl.ds(ebase + wbase, _PER_W)], ii_v)
        pltpu.sync_copy(jj_hbm.at[pl.ds(ebase + wbase, _PER_W)], jj_v)

        def issue(c, ri, rj, smi, smj):
            sl = pl.ds(c * _K, _K)
            pltpu.async_copy(gi_hbm.at[ii_v.at[sl]], ri, smi)
            pltpu.async_copy(gj_hbm.at[jj_v.at[sl]], rj, smj)

        def wait(ri, rj, smi, smj):
            pltpu.make_async_copy(gi_hbm.at[ii_v.at[pl.ds(0, _K)]], ri, smi).wait()
            pltpu.make_async_copy(gj_hbm.at[jj_v.at[pl.ds(0, _K)]], rj, smj).wait()

        def add_store(c, ri, rj):
            def row_add(r, rcarry):
                for cb in range(OUT_F // 16):
                    sl = pl.ds(cb * 16, 16)
                    ri[r, sl] = ri[r, sl] + rj[r, sl]
                return rcarry

            lax.fori_loop(0, _K, row_add, 0)
            pltpu.sync_copy(ri, out_hbm.at[pl.ds(wbase + c * _K, _K)])

        issue(0, ri0, rj0, smi0, smj0)

        def pair_body(t, carry):
            c0 = 2 * t
            issue(c0 + 1, ri1, rj1, smi1, smj1)
            wait(ri0, rj0, smi0, smj0)
            add_store(c0, ri0, rj0)
            issue(c0 + 2, ri0, rj0, smi0, smj0)
            wait(ri1, rj1, smi1, smj1)
            add_store(c0 + 1, ri1, rj1)
            return carry

        lax.fori_loop(0, (_CHUNKS - 1) // 2, pair_body, 0)
        wait(ri0, rj0, smi0, smj0)
        add_store(_CHUNKS - 1, ri0, rj0)

    return _edge_gather


def _edge_out_body(s_ref, rbf_ref, wr_ref, b_ref, o_ref):
    dn = (((1,), (1,)), ((), ()))
    t = lax.dot_general(rbf_ref[...], wr_ref[...], dn,
                        preferred_element_type=jnp.float32)
    x = s_ref[...] + t + b_ref[...]
    o_ref[...] = x / (1.0 + jnp.exp(-x))


def _edge_out_body_aliased(s_ref, rbf_ref, wr_ref, b_ref, prev_ref, o_ref):
    del prev_ref
    _edge_out_body(s_ref, rbf_ref, wr_ref, b_ref, o_ref)


def _edge_out_call(part, s, rbf, wr, b2d, prev=None):
    nb = _NB
    in_specs = [
        pl.BlockSpec((EB, OUT_F), lambda i: (i, 0)),
        pl.BlockSpec((EB, EDGE_F), lambda i, _p=part: (i + _p * nb, 0)),
        pl.BlockSpec((OUT_F, EDGE_F), lambda i: (0, 0)),
        pl.BlockSpec((1, OUT_F), lambda i: (0, 0)),
    ]
    args = [s, rbf, wr, b2d]
    body = _edge_out_body
    aliases = {}
    if prev is not None:
        in_specs.append(pl.BlockSpec(memory_space=pl.ANY))
        args.append(prev)
        body = _edge_out_body_aliased
        aliases = {4: 0}
    return pl.pallas_call(
        body,
        grid=(nb,),
        in_specs=in_specs,
        out_specs=pl.BlockSpec((EB, OUT_F), lambda i, _p=part: (i + _p * nb, 0)),
        out_shape=jax.ShapeDtypeStruct((E, OUT_F), jnp.float32),
        input_output_aliases=aliases,
    )(*args)


def kernel(z, rbf, idx_i, idx_j, emb, W, b):
    z2d = z.astype(jnp.int32).reshape(N, 1)
    emb_pad = jnp.pad(emb, ((0, ATOM_F - emb.shape[0]), (0, 0)))
    h, gi, gj = _node_call(z2d, emb_pad, W)
    ii = idx_i.astype(jnp.int32)
    jj = idx_j.astype(jnp.int32)
    wr = lax.slice(W, (0, 2 * ATOM_F), (OUT_F, 2 * ATOM_F + EDGE_F))
    b2d = b.reshape(1, OUT_F)
    ss = [_make_edge_gather(p)(gi, gj, ii, jj) for p in range(_NSPLIT)]
    m_ij = _edge_out_call(0, ss[0], rbf, wr, b2d)
    for p in range(1, _NSPLIT):
        m_ij = _edge_out_call(p, ss[p], rbf, wr, b2d, prev=m_ij)
    return (h, m_ij)


# speedup vs baseline: 4.5915x; 1.0203x over previous
import functools

import jax
import jax.numpy as jnp
from jax import lax
from jax.experimental import pallas as pl
from jax.experimental.pallas import tpu as pltpu
from jax.experimental.pallas import tpu_sc as plsc

N = 10000
E = 320000
ATOM_F = 128
EDGE_F = 16
OUT_F = 128

BN = 1000
EB = 2000

_NC = 2
_NS = 16
_NW = _NC * _NS
_NSPLIT = 2
_ESPLIT = E // _NSPLIT
_PER_W = _ESPLIT // _NW
_K = 200
_CHUNKS = _PER_W // _K
_NB = _ESPLIT // EB


def _node_body(z_ref, emb_ref, w_ref, h_ref, gi_ref, gj_ref):
    zm1 = z_ref[...] - 1
    col = lax.broadcasted_iota(jnp.int32, (BN, ATOM_F), 1)
    onehot = (zm1 == col).astype(jnp.float32)
    h = jnp.dot(onehot, emb_ref[...], preferred_element_type=jnp.float32)
    h_ref[...] = h
    wi = w_ref[:, 0:ATOM_F]
    wj = w_ref[:, ATOM_F:2 * ATOM_F]
    dn = (((1,), (1,)), ((), ()))
    gi_ref[...] = lax.dot_general(h, wi, dn, preferred_element_type=jnp.float32)
    gj_ref[...] = lax.dot_general(h, wj, dn, preferred_element_type=jnp.float32)


def _node_call(z2d, emb_pad, w):
    return pl.pallas_call(
        _node_body,
        grid=(N // BN,),
        in_specs=[
            pl.BlockSpec((BN, 1), lambda i: (i, 0)),
            pl.BlockSpec((ATOM_F, ATOM_F), lambda i: (0, 0)),
            pl.BlockSpec((OUT_F, 2 * ATOM_F + EDGE_F), lambda i: (0, 0)),
        ],
        out_specs=[pl.BlockSpec((BN, ATOM_F), lambda i: (i, 0))] * 3,
        out_shape=[jax.ShapeDtypeStruct((N, ATOM_F), jnp.float32)] * 3,
    )(z2d, emb_pad, w)


@functools.lru_cache(maxsize=None)
def _make_edge_gather(part):
    mesh = plsc.VectorSubcoreMesh(core_axis_name="c", subcore_axis_name="s")
    ebase = part * _ESPLIT

    @functools.partial(
        pl.kernel,
        mesh=mesh,
        out_type=jax.ShapeDtypeStruct((_ESPLIT, OUT_F), jnp.float32),
        scratch_types=[
            pltpu.VMEM((_PER_W,), jnp.int32),
            pltpu.VMEM((_PER_W,), jnp.int32),
            pltpu.VMEM((_K, OUT_F), jnp.float32),
            pltpu.VMEM((_K, OUT_F), jnp.float32),
            pltpu.VMEM((_K, OUT_F), jnp.float32),
            pltpu.VMEM((_K, OUT_F), jnp.float32),
            pltpu.SemaphoreType.DMA,
            pltpu.SemaphoreType.DMA,
            pltpu.SemaphoreType.DMA,
            pltpu.SemaphoreType.DMA,
        ],
    )
    def _edge_gather(gi_hbm, gj_hbm, ii_hbm, jj_hbm, out_hbm,
                     ii_v, jj_v, ri0, ri1, rj0, rj1,
                     smi0, smi1, smj0, smj1):
        wid = lax.axis_index("s") * _NC + lax.axis_index("c")
        wbase = wid * _PER_W
        pltpu.sync_copy(ii_hbm.at[pl.ds(ebase + wbase, _PER_W)], ii_v)
        pltpu.sync_copy(jj_hbm.at[pl.ds(ebase + wbase, _PER_W)], jj_v)

        def issue(c, ri, rj, smi, smj):
            sl = pl.ds(c * _K, _K)
            pltpu.async_copy(gi_hbm.at[ii_v.at[sl]], ri, smi)
            pltpu.async_copy(gj_hbm.at[jj_v.at[sl]], rj, smj)

        def wait(ri, rj, smi, smj):
            pltpu.make_async_copy(gi_hbm.at[ii_v.at[pl.ds(0, _K)]], ri, smi).wait()
            pltpu.make_async_copy(gj_hbm.at[jj_v.at[pl.ds(0, _K)]], rj, smj).wait()

        def add_store(c, ri, rj):
            def row_add(r, rcarry):
                for cb in range(OUT_F // 16):
                    sl = pl.ds(cb * 16, 16)
                    ri[r, sl] = ri[r, sl] + rj[r, sl]
                return rcarry

            lax.fori_loop(0, _K, row_add, 0)
            pltpu.sync_copy(ri, out_hbm.at[pl.ds(wbase + c * _K, _K)])

        issue(0, ri0, rj0, smi0, smj0)

        def pair_body(t, carry):
            c0 = 2 * t
            issue(c0 + 1, ri1, rj1, smi1, smj1)
            wait(ri0, rj0, smi0, smj0)
            add_store(c0, ri0, rj0)
            issue(c0 + 2, ri0, rj0, smi0, smj0)
            wait(ri1, rj1, smi1, smj1)
            add_store(c0 + 1, ri1, rj1)
            return carry

        lax.fori_loop(0, (_CHUNKS - 1) // 2, pair_body, 0)
        wait(ri0, rj0, smi0, smj0)
        add_store(_CHUNKS - 1, ri0, rj0)

    return _edge_gather


def _edge_out_body(s_ref, rbf_ref, wr_ref, b_ref, o_ref):
    dn = (((1,), (1,)), ((), ()))
    t = lax.dot_general(rbf_ref[...], wr_ref[...], dn,
                        preferred_element_type=jnp.float32)
    x = s_ref[...] + t + b_ref[...]
    o_ref[...] = x / (1.0 + jnp.exp(-x))


def _edge_out_body_aliased(s_ref, rbf_ref, wr_ref, b_ref, prev_ref, o_ref):
    del prev_ref
    _edge_out_body(s_ref, rbf_ref, wr_ref, b_ref, o_ref)


def _edge_out_call(part, s, rbf, wr, b2d, prev=None):
    nb = _NB
    in_specs = [
        pl.BlockSpec((EB, OUT_F), lambda i: (i, 0)),
        pl.BlockSpec((EB, EDGE_F), lambda i, _p=part: (i + _p * nb, 0)),
        pl.BlockSpec((OUT_F, EDGE_F), lambda i: (0, 0)),
        pl.BlockSpec((1, OUT_F), lambda i: (0, 0)),
    ]
    args = [s, rbf, wr, b2d]
    body = _edge_out_body
    aliases = {}
    if prev is not None:
        in_specs.append(pl.BlockSpec(memory_space=pl.ANY))
        args.append(prev)
        body = _edge_out_body_aliased
        aliases = {4: 0}
    return pl.pallas_call(
        body,
        grid=(nb,),
        in_specs=in_specs,
        out_specs=pl.BlockSpec((EB, OUT_F), lambda i, _p=part: (i + _p * nb, 0)),
        out_shape=jax.ShapeDtypeStruct((E, OUT_F), jnp.float32),
        input_output_aliases=aliases,
    )(*args)


def kernel(z, rbf, idx_i, idx_j, emb, W, b):
    z2d = z.astype(jnp.int32).reshape(N, 1)
    emb_pad = jnp.pad(emb, ((0, ATOM_F - emb.shape[0]), (0, 0)))
    h, gi, gj = _node_call(z2d, emb_pad, W)
    ii = idx_i.astype(jnp.int32)
    jj = idx_j.astype(jnp.int32)
    wr = lax.slice(W, (0, 2 * ATOM_F), (OUT_F, 2 * ATOM_F + EDGE_F))
    b2d = b.reshape(1, OUT_F)
    ss = [_make_edge_gather(p)(gi, gj, ii, jj) for p in range(_NSPLIT)]
    m_ij = _edge_out_call(0, ss[0], rbf, wr, b2d)
    for p in range(1, _NSPLIT):
        m_ij = _edge_out_call(p, ss[p], rbf, wr, b2d, prev=m_ij)
    return (h, m_ij)
